# baseline (speedup 1.0000x reference)
import jax
import jax.numpy as jnp
from jax.experimental import pallas as pl
from jax.experimental.pallas import tpu as pltpu

_N = 100000
_B = 64
_K = 16
_D = 128
_BM = 25088
_LAST = _N // _BM
_TAIL = _N - _LAST * _BM


def _probs_t(x, cent, c2):
    dot_t = jax.lax.dot_general(
        cent, x, (((1,), (1,)), ((), ())), preferred_element_type=jnp.float32
    )
    ones = jnp.ones((1, _D), dtype=jnp.float32)
    r2_t = jax.lax.dot_general(
        ones, x * x, (((1,), (1,)), ((), ())), preferred_element_type=jnp.float32
    )
    d2_t = r2_t + c2 - 2.0 * dot_t
    s_t = -jnp.sqrt(jnp.maximum(d2_t, 1e-12))
    m_t = jnp.max(s_t, axis=0, keepdims=True)
    e_t = jnp.exp(s_t - m_t)
    return e_t * (1.0 / jnp.sum(e_t, axis=0, keepdims=True))


def _body(res_ref, feats_ref, dc_ref, init_ref, out_ref):
    i = pl.program_id(0)
    cent = dc_ref[:] + init_ref[:]
    c2 = jnp.sum(cent * cent, axis=1, keepdims=True)

    @pl.when(i < _LAST)
    def _():
        out_ref[:] = _probs_t(res_ref[:], cent, c2)

    @pl.when(i == _LAST)
    def _():
        x = jnp.concatenate(
            [
                res_ref[0:_TAIL, :],
                feats_ref[:],
                jnp.zeros((_BM - _TAIL - _B, _D), jnp.float32),
            ],
            axis=0,
        )
        out_ref[:] = _probs_t(x, cent, c2)


def kernel(feats, reservoir_feats, delta_centroids, init_style):
    grid = _LAST + 1
    out_t = pl.pallas_call(
        _body,
        grid=(grid,),
        in_specs=[
            pl.BlockSpec((_BM, _D), lambda i: (i, 0)),
            pl.BlockSpec((_B, _D), lambda i: (0, 0)),
            pl.BlockSpec((_K, _D), lambda i: (0, 0)),
            pl.BlockSpec((1, _D), lambda i: (0, 0)),
        ],
        out_specs=pl.BlockSpec((_K, _BM), lambda i: (0, i)),
        out_shape=jax.ShapeDtypeStruct((_K, _N + _B), jnp.float32),
        compiler_params=pltpu.CompilerParams(
            dimension_semantics=(pltpu.PARALLEL,)
        ),
    )(reservoir_feats, feats, delta_centroids, init_style)
    return out_t.T


# speedup vs baseline: 1.0021x; 1.0021x over previous
import jax
import jax.numpy as jnp
from jax.experimental import pallas as pl
from jax.experimental.pallas import tpu as pltpu

_N = 100000
_B = 64
_K = 16
_D = 128
_BM = 24576
_LAST = _N // _BM
_TAIL = _N - _LAST * _BM


def _probs_t(x, cent, c2):
    dot_t = jax.lax.dot_general(
        cent, x, (((1,), (1,)), ((), ())), preferred_element_type=jnp.float32
    )
    ones = jnp.ones((1, _D), dtype=jnp.float32)
    r2_t = jax.lax.dot_general(
        ones, x * x, (((1,), (1,)), ((), ())), preferred_element_type=jnp.float32
    )
    d2_t = r2_t + c2 - 2.0 * dot_t
    s_t = -jnp.sqrt(jnp.maximum(d2_t, 1e-12))
    m_t = jnp.max(s_t, axis=0, keepdims=True)
    e_t = jnp.exp(s_t - m_t)
    return e_t * (1.0 / jnp.sum(e_t, axis=0, keepdims=True))


def _body(res_ref, feats_ref, dc_ref, init_ref, out_ref):
    i = pl.program_id(0)
    cent = dc_ref[:] + init_ref[:]
    c2 = jnp.sum(cent * cent, axis=1, keepdims=True)

    @pl.when(i < _LAST)
    def _():
        out_ref[:] = _probs_t(res_ref[:], cent, c2)

    @pl.when(i == _LAST)
    def _():
        x = jnp.concatenate(
            [
                res_ref[0:_TAIL, :],
                feats_ref[:],
                jnp.zeros((_BM - _TAIL - _B, _D), jnp.float32),
            ],
            axis=0,
        )
        out_ref[:] = _probs_t(x, cent, c2)


def kernel(feats, reservoir_feats, delta_centroids, init_style):
    grid = _LAST + 1
    out_t = pl.pallas_call(
        _body,
        grid=(grid,),
        in_specs=[
            pl.BlockSpec((_BM, _D), lambda i: (i, 0)),
            pl.BlockSpec((_B, _D), lambda i: (0, 0)),
            pl.BlockSpec((_K, _D), lambda i: (0, 0)),
            pl.BlockSpec((1, _D), lambda i: (0, 0)),
        ],
        out_specs=pl.BlockSpec((_K, _BM), lambda i: (0, i)),
        out_shape=jax.ShapeDtypeStruct((_K, _N + _B), jnp.float32),
        compiler_params=pltpu.CompilerParams(
            dimension_semantics=(pltpu.PARALLEL,)
        ),
    )(reservoir_feats, feats, delta_centroids, init_style)
    return out_t.T


# speedup vs baseline: 1.0057x; 1.0036x over previous
import jax
import jax.numpy as jnp
from jax.experimental import pallas as pl
from jax.experimental.pallas import tpu as pltpu

_N = 100000
_B = 64
_K = 16
_D = 128
_BM = 24576
_LAST = _N // _BM
_TAIL = _N - _LAST * _BM


def _probs_t(x, neg2cent, c2):
    neg2dot_t = jax.lax.dot_general(
        neg2cent, x, (((1,), (1,)), ((), ())), preferred_element_type=jnp.float32
    )
    ones = jnp.ones((1, _D), dtype=jnp.float32)
    r2_t = jax.lax.dot_general(
        ones, x * x, (((1,), (1,)), ((), ())), preferred_element_type=jnp.float32
    )
    d2_t = (r2_t + c2) + neg2dot_t
    s_t = -jnp.sqrt(jnp.maximum(d2_t, 1e-12))
    m_t = jnp.max(s_t, axis=0, keepdims=True)
    e_t = jnp.exp(s_t - m_t)
    return e_t * (1.0 / jnp.sum(e_t, axis=0, keepdims=True))


def _body(res_ref, feats_ref, dc_ref, init_ref, out_ref):
    i = pl.program_id(0)
    cent = dc_ref[:] + init_ref[:]
    c2 = jnp.sum(cent * cent, axis=1, keepdims=True)
    neg2cent = -2.0 * cent

    @pl.when(i < _LAST)
    def _():
        out_ref[:] = _probs_t(res_ref[:], neg2cent, c2)

    @pl.when(i == _LAST)
    def _():
        x = jnp.concatenate(
            [
                res_ref[0:_TAIL, :],
                feats_ref[:],
                jnp.zeros((_BM - _TAIL - _B, _D), jnp.float32),
            ],
            axis=0,
        )
        out_ref[:] = _probs_t(x, neg2cent, c2)


def kernel(feats, reservoir_feats, delta_centroids, init_style):
    grid = _LAST + 1
    out_t = pl.pallas_call(
        _body,
        grid=(grid,),
        in_specs=[
            pl.BlockSpec((_BM, _D), lambda i: (i, 0)),
            pl.BlockSpec((_B, _D), lambda i: (0, 0)),
            pl.BlockSpec((_K, _D), lambda i: (0, 0)),
            pl.BlockSpec((1, _D), lambda i: (0, 0)),
        ],
        out_specs=pl.BlockSpec((_K, _BM), lambda i: (0, i)),
        out_shape=jax.ShapeDtypeStruct((_K, _N + _B), jnp.float32),
        compiler_params=pltpu.CompilerParams(
            dimension_semantics=(pltpu.PARALLEL,)
        ),
    )(reservoir_feats, feats, delta_centroids, init_style)
    return out_t.T
